# baseline (speedup 1.0000x reference)
import functools

import jax
import jax.numpy as jnp
from jax import lax
from jax.experimental import pallas as pl
from jax.experimental.pallas import tpu as pltpu
from jax.experimental.pallas import tpu_sc as plsc

_ROWS = 16384
_COLS = 512
_BLOCK_ROWS = 4096
_WPAD = 9040


def _make_sc_lookup():
    mesh = plsc.VectorSubcoreMesh(core_axis_name="c", subcore_axis_name="s")

    @functools.partial(
        pl.kernel,
        mesh=mesh,
        out_type=jax.ShapeDtypeStruct((16,), jnp.float32),
        scratch_types=[
            pltpu.VMEM((16,), jnp.int32),
            pltpu.VMEM((16,), jnp.float32),
            pltpu.VMEM((16,), jnp.float32),
            pltpu.SemaphoreType.DMA,
        ],
    )
    def _lookup(w_hbm, idx_hbm, out_hbm, idx_v, vals_v, s_v, sem):
        cid = lax.axis_index("c")
        sid = lax.axis_index("s")

        @pl.when((cid == 0) & (sid == 0))
        def _():
            pltpu.sync_copy(idx_hbm, idx_v)
            pltpu.async_copy(w_hbm.at[idx_v], vals_v, sem).wait()
            s_v[...] = 1.0 / (1.0 + jnp.exp(-vals_v[...]))
            pltpu.sync_copy(s_v, out_hbm)

    return _lookup


_sc_lookup = _make_sc_lookup()


def _scale_kernel(s_ref, x_ref, out_ref):
    out_ref[...] = x_ref[...] * s_ref[0:1, 0:1]


def kernel(input_features, part_cls, obj_cls, W):
    p = jnp.asarray(part_cls, jnp.int32)
    o = jnp.asarray(obj_cls, jnp.int32)
    idx16 = jnp.full((16,), p * 95 + o, dtype=jnp.int32)
    w_flat = jnp.pad(W.reshape(-1), (0, _WPAD - 95 * 95))
    s16 = _sc_lookup(w_flat, idx16)

    grid = _ROWS // _BLOCK_ROWS
    return pl.pallas_call(
        _scale_kernel,
        grid=(grid,),
        in_specs=[
            pl.BlockSpec((1, 16), lambda i: (0, 0)),
            pl.BlockSpec((_BLOCK_ROWS, _COLS), lambda i: (i, 0)),
        ],
        out_specs=pl.BlockSpec((_BLOCK_ROWS, _COLS), lambda i: (i, 0)),
        out_shape=jax.ShapeDtypeStruct((_ROWS, _COLS), jnp.float32),
        compiler_params=pltpu.CompilerParams(
            dimension_semantics=("arbitrary",),
        ),
    )(s16.reshape(1, 16), input_features)


# speedup vs baseline: 1.6721x; 1.6721x over previous
import jax
import jax.numpy as jnp
from jax.experimental import pallas as pl
from jax.experimental.pallas import tpu as pltpu

_ROWS = 16384
_COLS = 512
_BLOCK_ROWS = 1024
_NBLK = _ROWS // _BLOCK_ROWS


def _scale_kernel(p_ref, o_ref, w_ref, x_hbm, out_hbm, bufs, in_sems, out_sems):
    rows = jax.lax.broadcasted_iota(jnp.int32, (95, 95), 0)
    cols = jax.lax.broadcasted_iota(jnp.int32, (95, 95), 1)
    hit = (rows == p_ref[0]) & (cols == o_ref[0])
    s = jax.nn.sigmoid(jnp.sum(jnp.where(hit, w_ref[...], 0.0)))

    in_copies = [
        pltpu.make_async_copy(
            x_hbm.at[pl.ds(i * _BLOCK_ROWS, _BLOCK_ROWS), :],
            bufs.at[i],
            in_sems.at[i],
        )
        for i in range(_NBLK)
    ]
    out_copies = [
        pltpu.make_async_copy(
            bufs.at[i],
            out_hbm.at[pl.ds(i * _BLOCK_ROWS, _BLOCK_ROWS), :],
            out_sems.at[i],
        )
        for i in range(_NBLK)
    ]
    for c in in_copies:
        c.start()
    for i in range(_NBLK):
        in_copies[i].wait()
        bufs[i] = bufs[i] * s
        out_copies[i].start()
    for c in out_copies:
        c.wait()


def kernel(input_features, part_cls, obj_cls, W):
    p = jnp.asarray(part_cls, jnp.int32).reshape(1)
    o = jnp.asarray(obj_cls, jnp.int32).reshape(1)
    w2d = W.reshape(95, 95)
    return pl.pallas_call(
        _scale_kernel,
        grid_spec=pltpu.PrefetchScalarGridSpec(
            num_scalar_prefetch=2,
            grid=(1,),
            in_specs=[
                pl.BlockSpec((95, 95), lambda i, p, o: (0, 0)),
                pl.BlockSpec(memory_space=pl.ANY),
            ],
            out_specs=pl.BlockSpec(memory_space=pl.ANY),
            scratch_shapes=[
                pltpu.VMEM((_NBLK, _BLOCK_ROWS, _COLS), jnp.float32),
                pltpu.SemaphoreType.DMA((_NBLK,)),
                pltpu.SemaphoreType.DMA((_NBLK,)),
            ],
        ),
        out_shape=jax.ShapeDtypeStruct((_ROWS, _COLS), jnp.float32),
        compiler_params=pltpu.CompilerParams(
            dimension_semantics=("arbitrary",),
        ),
    )(p, o, w2d, input_features)


# speedup vs baseline: 1.6881x; 1.0096x over previous
import jax
import jax.numpy as jnp
from jax.experimental import pallas as pl
from jax.experimental.pallas import tpu as pltpu

_ROWS = 16384
_COLS = 512
_BLOCK_ROWS = 2048
_NBLK = _ROWS // _BLOCK_ROWS


def _scale_kernel(p_ref, o_ref, w_ref, x_hbm, out_hbm, bufs, in_sems, out_sems):
    rows = jax.lax.broadcasted_iota(jnp.int32, (95, 95), 0)
    cols = jax.lax.broadcasted_iota(jnp.int32, (95, 95), 1)
    hit = (rows == p_ref[0]) & (cols == o_ref[0])
    s = jax.nn.sigmoid(jnp.sum(jnp.where(hit, w_ref[...], 0.0)))

    in_copies = [
        pltpu.make_async_copy(
            x_hbm.at[pl.ds(i * _BLOCK_ROWS, _BLOCK_ROWS), :],
            bufs.at[i],
            in_sems.at[i],
        )
        for i in range(_NBLK)
    ]
    out_copies = [
        pltpu.make_async_copy(
            bufs.at[i],
            out_hbm.at[pl.ds(i * _BLOCK_ROWS, _BLOCK_ROWS), :],
            out_sems.at[i],
        )
        for i in range(_NBLK)
    ]
    for c in in_copies:
        c.start()
    for i in range(_NBLK):
        in_copies[i].wait()
        bufs[i] = bufs[i] * s
        out_copies[i].start()
    for c in out_copies:
        c.wait()


def kernel(input_features, part_cls, obj_cls, W):
    p = jnp.asarray(part_cls, jnp.int32).reshape(1)
    o = jnp.asarray(obj_cls, jnp.int32).reshape(1)
    w2d = W.reshape(95, 95)
    return pl.pallas_call(
        _scale_kernel,
        grid_spec=pltpu.PrefetchScalarGridSpec(
            num_scalar_prefetch=2,
            grid=(1,),
            in_specs=[
                pl.BlockSpec((95, 95), lambda i, p, o: (0, 0)),
                pl.BlockSpec(memory_space=pl.ANY),
            ],
            out_specs=pl.BlockSpec(memory_space=pl.ANY),
            scratch_shapes=[
                pltpu.VMEM((_NBLK, _BLOCK_ROWS, _COLS), jnp.float32),
                pltpu.SemaphoreType.DMA((_NBLK,)),
                pltpu.SemaphoreType.DMA((_NBLK,)),
            ],
        ),
        out_shape=jax.ShapeDtypeStruct((_ROWS, _COLS), jnp.float32),
        compiler_params=pltpu.CompilerParams(
            dimension_semantics=("arbitrary",),
        ),
    )(p, o, w2d, input_features)


# speedup vs baseline: 1.7072x; 1.0113x over previous
import jax
import jax.numpy as jnp
from jax.experimental import pallas as pl
from jax.experimental.pallas import tpu as pltpu

_ROWS = 16384
_COLS = 512
_BLOCK_ROWS = 4096
_NBLK = _ROWS // _BLOCK_ROWS


def _scale_kernel(p_ref, o_ref, w_ref, x_hbm, out_hbm, bufs, in_sems, out_sems):
    rows = jax.lax.broadcasted_iota(jnp.int32, (95, 95), 0)
    cols = jax.lax.broadcasted_iota(jnp.int32, (95, 95), 1)
    hit = (rows == p_ref[0]) & (cols == o_ref[0])
    s = jax.nn.sigmoid(jnp.sum(jnp.where(hit, w_ref[...], 0.0)))

    in_copies = [
        pltpu.make_async_copy(
            x_hbm.at[pl.ds(i * _BLOCK_ROWS, _BLOCK_ROWS), :],
            bufs.at[i],
            in_sems.at[i],
        )
        for i in range(_NBLK)
    ]
    out_copies = [
        pltpu.make_async_copy(
            bufs.at[i],
            out_hbm.at[pl.ds(i * _BLOCK_ROWS, _BLOCK_ROWS), :],
            out_sems.at[i],
        )
        for i in range(_NBLK)
    ]
    for c in in_copies:
        c.start()
    for i in range(_NBLK):
        in_copies[i].wait()
        bufs[i] = bufs[i] * s
        out_copies[i].start()
    for c in out_copies:
        c.wait()


def kernel(input_features, part_cls, obj_cls, W):
    p = jnp.asarray(part_cls, jnp.int32).reshape(1)
    o = jnp.asarray(obj_cls, jnp.int32).reshape(1)
    w2d = W.reshape(95, 95)
    return pl.pallas_call(
        _scale_kernel,
        grid_spec=pltpu.PrefetchScalarGridSpec(
            num_scalar_prefetch=2,
            grid=(1,),
            in_specs=[
                pl.BlockSpec((95, 95), lambda i, p, o: (0, 0)),
                pl.BlockSpec(memory_space=pl.ANY),
            ],
            out_specs=pl.BlockSpec(memory_space=pl.ANY),
            scratch_shapes=[
                pltpu.VMEM((_NBLK, _BLOCK_ROWS, _COLS), jnp.float32),
                pltpu.SemaphoreType.DMA((_NBLK,)),
                pltpu.SemaphoreType.DMA((_NBLK,)),
            ],
        ),
        out_shape=jax.ShapeDtypeStruct((_ROWS, _COLS), jnp.float32),
        compiler_params=pltpu.CompilerParams(
            dimension_semantics=("arbitrary",),
        ),
    )(p, o, w2d, input_features)


# speedup vs baseline: 1.7422x; 1.0205x over previous
import jax
import jax.numpy as jnp
from jax.experimental import pallas as pl
from jax.experimental.pallas import tpu as pltpu

_ROWS = 16384
_COLS = 512
_BLOCK_ROWS = 8192
_NBLK = _ROWS // _BLOCK_ROWS


def _scale_kernel(p_ref, o_ref, w_ref, x_hbm, out_hbm, bufs, in_sems, out_sems):
    rows = jax.lax.broadcasted_iota(jnp.int32, (95, 95), 0)
    cols = jax.lax.broadcasted_iota(jnp.int32, (95, 95), 1)
    hit = (rows == p_ref[0]) & (cols == o_ref[0])
    s = jax.nn.sigmoid(jnp.sum(jnp.where(hit, w_ref[...], 0.0)))

    in_copies = [
        pltpu.make_async_copy(
            x_hbm.at[pl.ds(i * _BLOCK_ROWS, _BLOCK_ROWS), :],
            bufs.at[i],
            in_sems.at[i],
        )
        for i in range(_NBLK)
    ]
    out_copies = [
        pltpu.make_async_copy(
            bufs.at[i],
            out_hbm.at[pl.ds(i * _BLOCK_ROWS, _BLOCK_ROWS), :],
            out_sems.at[i],
        )
        for i in range(_NBLK)
    ]
    for c in in_copies:
        c.start()
    for i in range(_NBLK):
        in_copies[i].wait()
        bufs[i] = bufs[i] * s
        out_copies[i].start()
    for c in out_copies:
        c.wait()


def kernel(input_features, part_cls, obj_cls, W):
    p = jnp.asarray(part_cls, jnp.int32).reshape(1)
    o = jnp.asarray(obj_cls, jnp.int32).reshape(1)
    w2d = W.reshape(95, 95)
    return pl.pallas_call(
        _scale_kernel,
        grid_spec=pltpu.PrefetchScalarGridSpec(
            num_scalar_prefetch=2,
            grid=(1,),
            in_specs=[
                pl.BlockSpec((95, 95), lambda i, p, o: (0, 0)),
                pl.BlockSpec(memory_space=pl.ANY),
            ],
            out_specs=pl.BlockSpec(memory_space=pl.ANY),
            scratch_shapes=[
                pltpu.VMEM((_NBLK, _BLOCK_ROWS, _COLS), jnp.float32),
                pltpu.SemaphoreType.DMA((_NBLK,)),
                pltpu.SemaphoreType.DMA((_NBLK,)),
            ],
        ),
        out_shape=jax.ShapeDtypeStruct((_ROWS, _COLS), jnp.float32),
        compiler_params=pltpu.CompilerParams(
            dimension_semantics=("arbitrary",),
        ),
    )(p, o, w2d, input_features)


# speedup vs baseline: 1.7641x; 1.0126x over previous
import jax
import jax.numpy as jnp
from jax.experimental import pallas as pl
from jax.experimental.pallas import tpu as pltpu

_ROWS = 16384
_COLS = 512
_BLOCK_ROWS = 4096


def _scale_kernel(p_ref, o_ref, w_ref, x_ref, out_ref, s_ref):
    @pl.when(pl.program_id(0) == 0)
    def _():
        rows = jax.lax.broadcasted_iota(jnp.int32, (95, 95), 0)
        cols = jax.lax.broadcasted_iota(jnp.int32, (95, 95), 1)
        hit = (rows == p_ref[0]) & (cols == o_ref[0])
        w = jnp.sum(jnp.where(hit, w_ref[...], 0.0))
        s_ref[0] = jax.nn.sigmoid(w)

    out_ref[...] = x_ref[...] * s_ref[0]


def kernel(input_features, part_cls, obj_cls, W):
    p = jnp.asarray(part_cls, jnp.int32).reshape(1)
    o = jnp.asarray(obj_cls, jnp.int32).reshape(1)
    w2d = W.reshape(95, 95)
    grid = _ROWS // _BLOCK_ROWS
    return pl.pallas_call(
        _scale_kernel,
        grid_spec=pltpu.PrefetchScalarGridSpec(
            num_scalar_prefetch=2,
            grid=(grid,),
            in_specs=[
                pl.BlockSpec((95, 95), lambda i, p, o: (0, 0)),
                pl.BlockSpec((_BLOCK_ROWS, _COLS), lambda i, p, o: (i, 0)),
            ],
            out_specs=pl.BlockSpec((_BLOCK_ROWS, _COLS), lambda i, p, o: (i, 0)),
            scratch_shapes=[pltpu.SMEM((1,), jnp.float32)],
        ),
        out_shape=jax.ShapeDtypeStruct((_ROWS, _COLS), jnp.float32),
        compiler_params=pltpu.CompilerParams(
            dimension_semantics=("arbitrary",),
        ),
    )(p, o, w2d, input_features)
